# baseline (speedup 1.0000x reference)
import functools

import jax
import jax.numpy as jnp
from jax import lax
from jax.experimental import pallas as pl
from jax.experimental.pallas import tpu as pltpu
from jax.experimental.pallas import tpu_sc as plsc

D = 128
NC = 2
NS = 16
NW = NC * NS
BLK = 128


def _sc_segment_sum(table_rows, acc_rows, nb0, nb1):
    nb = max(nb0, nb1)
    rpt = acc_rows // NS
    mesh = plsc.VectorSubcoreMesh(
        core_axis_name="c", subcore_axis_name="s", num_cores=NC, num_subcores=NS
    )

    @functools.partial(
        pl.kernel,
        out_type=jax.ShapeDtypeStruct((NC, acc_rows, D), jnp.float32),
        mesh=mesh,
        scratch_types=[
            pltpu.VMEM((nb, BLK), jnp.int32),
            pltpu.VMEM((nb, BLK), jnp.int32),
            pltpu.VMEM((BLK, D), jnp.float32),
            pltpu.VMEM_SHARED((acc_rows, D), jnp.float32),
            pltpu.SemaphoreType.DMA,
            pltpu.SemaphoreType.DMA,
        ],
    )
    def sc_kernel(tab_hbm, gidx_hbm, sidx_hbm, zeros_hbm, out_hbm,
                  gidx_v, sidx_v, rows_v, acc_s, gsem, ssem):
        c = lax.axis_index("c")
        s = lax.axis_index("s")
        wid = c * NS + s
        r0 = s * rpt
        pltpu.sync_copy(zeros_hbm.at[pl.ds(r0, rpt)], acc_s.at[pl.ds(r0, rpt)])
        pltpu.sync_copy(gidx_hbm.at[wid], gidx_v)
        pltpu.sync_copy(sidx_hbm.at[wid], sidx_v)
        plsc.subcore_barrier()

        def body(b, carry):
            pltpu.async_copy(tab_hbm.at[gidx_v.at[b]], rows_v, gsem).wait()
            pltpu.async_copy(rows_v, acc_s.at[sidx_v.at[b]], ssem, add=True).wait()
            return carry

        lax.fori_loop(0, jnp.where(c == 0, nb0, nb1), body, 0)
        plsc.subcore_barrier()
        pltpu.sync_copy(acc_s.at[pl.ds(r0, rpt)], out_hbm.at[c, pl.ds(r0, rpt)])

    return sc_kernel


def _dense_mid(p_ref, wlT_ref, bl_ref, whT_ref, wt_ref, cf_ref, y_ref):
    hc = p_ref[0] + p_ref[1]
    cembs = jnp.maximum(
        jnp.dot(hc, wlT_ref[...], preferred_element_type=jnp.float32)
        + bl_ref[...], 0.0)
    y_ref[...] = (
        jnp.dot(cembs, whT_ref[...], preferred_element_type=jnp.float32)
        + cf_ref[...] * wt_ref[...])


def _dense_out(p_ref, bo_ref, o_ref, n_out):
    o_ref[...] = jnp.maximum(p_ref[0, :n_out] + p_ref[1, :n_out] + bo_ref[...], 0.0)


def kernel(literal_feat, clause_feat, W_l2c, b_l2c, W_c2l, b_c2l, lit_idx, clause_idx):
    n_lit, _ = literal_feat.shape
    n_clause = clause_feat.shape[0]
    e = lit_idx.shape[0]

    c_pad = ((n_clause + 1 + NS * 8 - 1) // (NS * 8)) * (NS * 8)
    l_pad = ((n_lit + 1 + NS * 8 - 1) // (NS * 8)) * (NS * 8)

    def split_idx(idx, frac0, fill):
        tot = -(-e // BLK)
        nb0 = max(1, int(round(tot * frac0 / NS)))
        nb1 = -(-(tot - NS * nb0) // NS)
        nb_max = max(nb0, nb1)
        idx = idx.astype(jnp.int32)
        pad = NS * (nb0 + nb1) * BLK - e
        idxp = jnp.concatenate([idx, jnp.full((pad,), fill, jnp.int32)])
        h0 = idxp[:NS * nb0 * BLK].reshape(NS, nb0, BLK)
        h1 = idxp[NS * nb0 * BLK:].reshape(NS, nb1, BLK)
        f = jnp.full((NS, nb_max, BLK), fill, jnp.int32)
        h0 = jnp.concatenate([h0, f[:, nb0:]], axis=1)
        h1 = jnp.concatenate([h1, f[:, nb1:]], axis=1)
        return jnp.concatenate([h0, h1], axis=0), nb0, nb1

    F1, F2 = 0.56, 0.60
    li1, nb0_1, nb1_1 = split_idx(lit_idx, F1, n_lit)
    ci1, _, _ = split_idx(clause_idx, F1, n_clause)
    li2, nb0_2, nb1_2 = split_idx(lit_idx, F2, n_lit)
    ci2, _, _ = split_idx(clause_idx, F2, n_clause)

    lit_tab = jnp.concatenate(
        [literal_feat, jnp.zeros((16, D), jnp.float32)], axis=0)
    zeros = jnp.zeros((l_pad, D), jnp.float32)

    part_c = _sc_segment_sum(lit_tab.shape[0], c_pad, nb0_1, nb1_1)(
        lit_tab, li1, ci1, zeros)

    wlT = W_l2c.T
    whT = W_c2l[:, :D].T
    wt = W_c2l[:, D].reshape(1, D)
    cf = jnp.concatenate(
        [clause_feat.astype(jnp.float32),
         jnp.zeros((c_pad - n_clause, 1), jnp.float32)], axis=0)
    y = pl.pallas_call(
        _dense_mid,
        out_shape=jax.ShapeDtypeStruct((c_pad, D), jnp.float32),
    )(part_c, wlT, b_l2c.reshape(1, D), whT, wt, cf)

    part_l = _sc_segment_sum(c_pad, l_pad, nb0_2, nb1_2)(y, ci2, li2, zeros)

    lembs = pl.pallas_call(
        functools.partial(_dense_out, n_out=n_lit),
        out_shape=jax.ShapeDtypeStruct((n_lit, D), jnp.float32),
    )(part_l, b_c2l.reshape(1, D))
    return lembs


# speedup vs baseline: 1.0144x; 1.0144x over previous
import functools

import jax
import jax.numpy as jnp
from jax import lax
from jax.experimental import pallas as pl
from jax.experimental.pallas import tpu as pltpu
from jax.experimental.pallas import tpu_sc as plsc

D = 128
NC = 2
NS = 16
NW = NC * NS
BLK = 128


def _sc_segment_sum(table_rows, acc_rows, nb0, nb1):
    nb = max(nb0, nb1)
    rpt = acc_rows // NS
    mesh = plsc.VectorSubcoreMesh(
        core_axis_name="c", subcore_axis_name="s", num_cores=NC, num_subcores=NS
    )

    @functools.partial(
        pl.kernel,
        out_type=jax.ShapeDtypeStruct((NC, acc_rows, D), jnp.float32),
        mesh=mesh,
        scratch_types=[
            pltpu.VMEM((nb, BLK), jnp.int32),
            pltpu.VMEM((nb, BLK), jnp.int32),
            pltpu.VMEM((BLK, D), jnp.float32),
            pltpu.VMEM_SHARED((acc_rows, D), jnp.float32),
            pltpu.SemaphoreType.DMA,
            pltpu.SemaphoreType.DMA,
        ],
    )
    def sc_kernel(tab_hbm, gidx_hbm, sidx_hbm, zeros_hbm, out_hbm,
                  gidx_v, sidx_v, rows_v, acc_s, gsem, ssem):
        c = lax.axis_index("c")
        s = lax.axis_index("s")
        wid = c * NS + s
        r0 = s * rpt
        pltpu.sync_copy(zeros_hbm.at[pl.ds(r0, rpt)], acc_s.at[pl.ds(r0, rpt)])
        pltpu.sync_copy(gidx_hbm.at[wid], gidx_v)
        pltpu.sync_copy(sidx_hbm.at[wid], sidx_v)
        plsc.subcore_barrier()

        def body(b, carry):
            pltpu.async_copy(tab_hbm.at[gidx_v.at[b]], rows_v, gsem).wait()
            pltpu.async_copy(rows_v, acc_s.at[sidx_v.at[b]], ssem, add=True).wait()
            return carry

        lax.fori_loop(0, jnp.where(c == 0, nb0, nb1), body, 0)
        plsc.subcore_barrier()
        pltpu.sync_copy(acc_s.at[pl.ds(r0, rpt)], out_hbm.at[c, pl.ds(r0, rpt)])

    return sc_kernel


def _dense_mid(p_ref, wlT_ref, bl_ref, whT_ref, wt_ref, cf_ref, y_ref):
    hc = p_ref[0] + p_ref[1]
    cembs = jnp.maximum(
        jnp.dot(hc, wlT_ref[...], preferred_element_type=jnp.float32)
        + bl_ref[...], 0.0)
    y_ref[...] = (
        jnp.dot(cembs, whT_ref[...], preferred_element_type=jnp.float32)
        + cf_ref[...] * wt_ref[...])


def _dense_out(p_ref, bo_ref, o_ref, n_out):
    o_ref[...] = jnp.maximum(p_ref[0, :n_out] + p_ref[1, :n_out] + bo_ref[...], 0.0)


def kernel(literal_feat, clause_feat, W_l2c, b_l2c, W_c2l, b_c2l, lit_idx, clause_idx):
    n_lit, _ = literal_feat.shape
    n_clause = clause_feat.shape[0]
    e = lit_idx.shape[0]

    c_pad = ((n_clause + 1 + NS * 8 - 1) // (NS * 8)) * (NS * 8)
    l_pad = ((n_lit + 1 + NS * 8 - 1) // (NS * 8)) * (NS * 8)

    def split_idx(idx, frac0, fill):
        tot = -(-e // BLK)
        nb0 = max(1, int(round(tot * frac0 / NS)))
        nb1 = -(-(tot - NS * nb0) // NS)
        nb_max = max(nb0, nb1)
        idx = idx.astype(jnp.int32)
        pad = NS * (nb0 + nb1) * BLK - e
        idxp = jnp.concatenate([idx, jnp.full((pad,), fill, jnp.int32)])
        h0 = idxp[:NS * nb0 * BLK].reshape(NS, nb0, BLK)
        h1 = idxp[NS * nb0 * BLK:].reshape(NS, nb1, BLK)
        f = jnp.full((NS, nb_max, BLK), fill, jnp.int32)
        h0 = jnp.concatenate([h0, f[:, nb0:]], axis=1)
        h1 = jnp.concatenate([h1, f[:, nb1:]], axis=1)
        return jnp.concatenate([h0, h1], axis=0), nb0, nb1

    F1, F2 = 0.58, 0.62
    li1, nb0_1, nb1_1 = split_idx(lit_idx, F1, n_lit)
    ci1, _, _ = split_idx(clause_idx, F1, n_clause)
    li2, nb0_2, nb1_2 = split_idx(lit_idx, F2, n_lit)
    ci2, _, _ = split_idx(clause_idx, F2, n_clause)

    lit_tab = jnp.concatenate(
        [literal_feat, jnp.zeros((16, D), jnp.float32)], axis=0)
    zeros = jnp.zeros((l_pad, D), jnp.float32)

    part_c = _sc_segment_sum(lit_tab.shape[0], c_pad, nb0_1, nb1_1)(
        lit_tab, li1, ci1, zeros)

    wlT = W_l2c.T
    whT = W_c2l[:, :D].T
    wt = W_c2l[:, D].reshape(1, D)
    cf = jnp.concatenate(
        [clause_feat.astype(jnp.float32),
         jnp.zeros((c_pad - n_clause, 1), jnp.float32)], axis=0)
    y = pl.pallas_call(
        _dense_mid,
        out_shape=jax.ShapeDtypeStruct((c_pad, D), jnp.float32),
    )(part_c, wlT, b_l2c.reshape(1, D), whT, wt, cf)

    part_l = _sc_segment_sum(c_pad, l_pad, nb0_2, nb1_2)(y, ci2, li2, zeros)

    lembs = pl.pallas_call(
        functools.partial(_dense_out, n_out=n_lit),
        out_shape=jax.ShapeDtypeStruct((n_lit, D), jnp.float32),
    )(part_l, b_c2l.reshape(1, D))
    return lembs
